# baseline (speedup 1.0000x reference)
import jax
import jax.numpy as jnp
import numpy as np
from jax import lax
from jax.experimental import pallas as pl
from jax.experimental.pallas import tpu as pltpu
from jax.experimental.pallas import tpu_sc as plsc

N = 10000
E = 320000
D = 128

NC = 2
NS = 16
NW = NC * NS

N_PAD = 10240
RPT = N_PAD // NS

CHUNK = 112
NBUF = 3
NCH = 96
STG = 24
TOT_CH = NW * NCH
E_PAD = TOT_CH * CHUNK

_PAD_N = E_PAD - E
_PAD_SRC = np.arange(_PAD_N, dtype=np.int32) % N
_PAD_DST = (N + np.arange(_PAD_N, dtype=np.int32) % (N_PAD - N)).astype(np.int32)

_mesh = plsc.VectorSubcoreMesh(core_axis_name="c", subcore_axis_name="s")


EPT = E // NW


def _deg_body(dst_hbm, deg_out, hist, dbuf, hbuf, obuf, histall):
    c = lax.axis_index("c")
    s = lax.axis_index("s")
    wid = c * NS + s

    zeros16 = jnp.zeros((16,), jnp.float32)
    ones16 = jnp.ones((16,), jnp.float32)

    def zero_hist(i, _):
        hist[pl.ds(i * 16, 16)] = zeros16
        return _

    lax.fori_loop(0, N_PAD // 16, zero_hist, None)

    pltpu.sync_copy(dst_hbm.at[pl.ds(wid * EPT, EPT)], dbuf)

    def outer(r, _):
        idx = dbuf[pl.ds(r * 16, 16)]
        plsc.addupdate_scatter(hist, [idx], ones16)
        return _

    lax.fori_loop(0, EPT // 16, outer, None)

    pltpu.sync_copy(hist, histall.at[s])
    plsc.subcore_barrier()
    pltpu.sync_copy(histall.at[:, pl.ds(s * RPT, RPT)], hbuf)

    def red(i, _):
        acc = hbuf[0, pl.ds(i * 16, 16)]
        for k in range(1, NS):
            acc = acc + hbuf[k, pl.ds(i * 16, 16)]
        obuf[pl.ds(i * 16, 16)] = acc
        return _

    lax.fori_loop(0, RPT // 16, red, None)
    pltpu.sync_copy(obuf, deg_out.at[c, pl.ds(s * RPT, RPT)])


_deg_call = pl.kernel(
    _deg_body,
    out_type=jax.ShapeDtypeStruct((NC, N_PAD), jnp.float32),
    mesh=_mesh,
    compiler_params=pltpu.CompilerParams(needs_layout_passes=False),
    scratch_types=[
        pltpu.VMEM((N_PAD,), jnp.float32),
        pltpu.VMEM((EPT,), jnp.int32),
        pltpu.VMEM((NS, RPT), jnp.float32),
        pltpu.VMEM((RPT,), jnp.float32),
        pltpu.VMEM_SHARED((NS, N_PAD), jnp.float32),
    ],
)


def _agg_body(hs_hbm, src_hbm, dst_hbm, part_out,
              sidx, didx, rows, agg, gsem, ssem):
    c = lax.axis_index("c")
    s = lax.axis_index("s")
    wid = c * NS + s

    zeros16 = jnp.zeros((16,), jnp.float32)

    def zero_row(i, _):
        for t in range(NBUF):
            for k in range(D // 16):
                rows[t, i, pl.ds(k * 16, 16)] = zeros16
        return _

    lax.fori_loop(0, CHUNK, zero_row, None)
    for k in range(RPT // CHUNK):
        pltpu.sync_copy(rows.at[0], agg.at[pl.ds(s * RPT + k * CHUNK, CHUNK)])
    rem = RPT - (RPT // CHUNK) * CHUNK
    if rem:
        pltpu.sync_copy(rows.at[0].at[pl.ds(0, rem)],
                        agg.at[pl.ds(s * RPT + RPT - rem, rem)])
    plsc.subcore_barrier()

    def gstart(k, t):
        pltpu.async_copy(hs_hbm.at[sidx.at[k]], rows.at[t], gsem)

    def gwait(k, t):
        pltpu.make_async_copy(hs_hbm.at[sidx.at[k]], rows.at[t], gsem).wait()

    def sstart(k, t):
        return pltpu.async_copy(rows.at[t], agg.at[didx.at[k]], ssem,
                                add=True)

    for h in range(NCH // STG):
        row0 = wid * NCH + h * STG
        pltpu.sync_copy(src_hbm.at[pl.ds(row0, STG)], sidx)
        pltpu.sync_copy(dst_hbm.at[pl.ds(row0, STG)], didx)

        for t in range(NBUF):
            gstart(t, t)

        def group(i, _):
            k0 = i * NBUF
            sdescs = []
            for t in range(NBUF):
                gwait(k0 + t, t)
                sdescs.append(sstart(k0 + t, t))
            for t in range(NBUF):
                sdescs[t].wait()

                @pl.when(k0 + NBUF + t < STG)
                def _():
                    gstart(k0 + NBUF + t, t)

            return _

        lax.fori_loop(0, STG // NBUF, group, None)

    plsc.subcore_barrier()

    for k in range(RPT // CHUNK):
        r0 = s * RPT + k * CHUNK
        pltpu.sync_copy(agg.at[pl.ds(r0, CHUNK)], part_out.at[c, pl.ds(r0, CHUNK)])
    if RPT % CHUNK:
        r0 = s * RPT + (RPT // CHUNK) * CHUNK
        pltpu.sync_copy(agg.at[pl.ds(r0, RPT % CHUNK)],
                        part_out.at[c, pl.ds(r0, RPT % CHUNK)])


_agg_call = pl.kernel(
    _agg_body,
    out_type=jax.ShapeDtypeStruct((NC, N_PAD, D), jnp.float32),
    mesh=_mesh,
    scratch_types=[
        pltpu.VMEM((STG, CHUNK), jnp.int32),
        pltpu.VMEM((STG, CHUNK), jnp.int32),
        pltpu.VMEM((NBUF, CHUNK, D), jnp.float32),
        pltpu.VMEM_SHARED((N_PAD, D), jnp.float32),
        pltpu.SemaphoreType.DMA,
        pltpu.SemaphoreType.DMA,
    ],
)

_TC_R = 1024


def _matmul(x_ref, w_ref, h_ref):
    h_ref[...] = jnp.dot(x_ref[...], w_ref[...],
                         preferred_element_type=jnp.float32)


def _phase_a(h_ref, degp_ref, hs_ref, dinv_ref):
    deg = degp_ref[0] + degp_ref[1] + 1.0
    dinv = lax.rsqrt(deg)
    hs_ref[...] = h_ref[...] * dinv[:, None]
    dinv_ref[...] = dinv[:, None]


def _phase_b(agg_ref, hs_ref, dinv_ref, b_ref, w_ref, hs2_ref):
    a = agg_ref[0] + agg_ref[1]
    dinv = dinv_ref[...]
    h1 = jnp.maximum((a + hs_ref[...]) * dinv + b_ref[...], 0.0)
    h2 = jnp.dot(h1, w_ref[...], preferred_element_type=jnp.float32)
    hs2_ref[...] = h2 * dinv


def _phase_c(agg_ref, hs_ref, dinv_ref, b_ref, out_ref):
    a = agg_ref[0] + agg_ref[1]
    out_ref[...] = (a + hs_ref[...]) * dinv_ref[...] + b_ref[...]


_GRID = N_PAD // _TC_R

_matmul_call = pl.pallas_call(
    _matmul,
    grid=(_GRID,),
    in_specs=[
        pl.BlockSpec((_TC_R, D), lambda i: (i, 0)),
        pl.BlockSpec((D, D), lambda i: (0, 0)),
    ],
    out_specs=pl.BlockSpec((_TC_R, D), lambda i: (i, 0)),
    out_shape=jax.ShapeDtypeStruct((N_PAD, D), jnp.float32),
)

_phase_a_call = pl.pallas_call(
    _phase_a,
    grid=(_GRID,),
    in_specs=[
        pl.BlockSpec((_TC_R, D), lambda i: (i, 0)),
        pl.BlockSpec((NC, _TC_R), lambda i: (0, i)),
    ],
    out_specs=[
        pl.BlockSpec((_TC_R, D), lambda i: (i, 0)),
        pl.BlockSpec((_TC_R, 1), lambda i: (i, 0)),
    ],
    out_shape=[
        jax.ShapeDtypeStruct((N_PAD, D), jnp.float32),
        jax.ShapeDtypeStruct((N_PAD, 1), jnp.float32),
    ],
)

_phase_b_call = pl.pallas_call(
    _phase_b,
    grid=(_GRID,),
    in_specs=[
        pl.BlockSpec((NC, _TC_R, D), lambda i: (0, i, 0)),
        pl.BlockSpec((_TC_R, D), lambda i: (i, 0)),
        pl.BlockSpec((_TC_R, 1), lambda i: (i, 0)),
        pl.BlockSpec((D,), lambda i: (0,)),
        pl.BlockSpec((D, D), lambda i: (0, 0)),
    ],
    out_specs=pl.BlockSpec((_TC_R, D), lambda i: (i, 0)),
    out_shape=jax.ShapeDtypeStruct((N_PAD, D), jnp.float32),
)

_TC_RC = 2000

_phase_c_call = pl.pallas_call(
    _phase_c,
    grid=(N // _TC_RC,),
    in_specs=[
        pl.BlockSpec((NC, _TC_RC, D), lambda i: (0, i, 0)),
        pl.BlockSpec((_TC_RC, D), lambda i: (i, 0)),
        pl.BlockSpec((_TC_RC, 1), lambda i: (i, 0)),
        pl.BlockSpec((D,), lambda i: (0,)),
    ],
    out_specs=pl.BlockSpec((_TC_RC, D), lambda i: (i, 0)),
    out_shape=jax.ShapeDtypeStruct((N, D), jnp.float32),
)


@jax.jit
def kernel(x, edge_index, W1, b1, W2, b2):
    src = edge_index[0]
    dst = edge_index[1]
    src_p = jnp.concatenate([src, _PAD_SRC]).reshape(TOT_CH, CHUNK)
    dst_p = jnp.concatenate([dst, _PAD_DST]).reshape(TOT_CH, CHUNK)
    x_p = jnp.pad(x, ((0, N_PAD - N), (0, 0)))

    h1 = _matmul_call(x_p, W1)
    degp = _deg_call(dst)
    hs1, dinv = _phase_a_call(h1, degp)
    agg1 = _agg_call(hs1, src_p, dst_p)
    hs2 = _phase_b_call(agg1, hs1, dinv, b1, W2)
    agg2 = _agg_call(hs2, src_p, dst_p)
    return _phase_c_call(agg2, hs2, dinv, b2)


# speedup vs baseline: 1.0432x; 1.0432x over previous
import jax
import jax.numpy as jnp
import numpy as np
from jax import lax
from jax.experimental import pallas as pl
from jax.experimental.pallas import tpu as pltpu
from jax.experimental.pallas import tpu_sc as plsc

N = 10000
E = 320000
D = 128

NC = 2
NS = 16
NW = NC * NS

N_PAD = 10240
RPT = N_PAD // NS

CHUNK = 128
NBUF = 2
NCH = 80
STG = 40
TOT_CH = NW * NCH
E_PAD = TOT_CH * CHUNK

_PAD_N = E_PAD - E
_PAD_SRC = np.arange(_PAD_N, dtype=np.int32) % N
_PAD_DST = (N + np.arange(_PAD_N, dtype=np.int32) % (N_PAD - N)).astype(np.int32)

_mesh = plsc.VectorSubcoreMesh(core_axis_name="c", subcore_axis_name="s")


EPT = E // NW


def _deg_body(dst_hbm, deg_out, hist, dbuf, hbuf, obuf, histall):
    c = lax.axis_index("c")
    s = lax.axis_index("s")
    wid = c * NS + s

    zeros16 = jnp.zeros((16,), jnp.float32)
    ones16 = jnp.ones((16,), jnp.float32)

    def zero_hist(i, _):
        hist[pl.ds(i * 16, 16)] = zeros16
        return _

    lax.fori_loop(0, N_PAD // 16, zero_hist, None)

    pltpu.sync_copy(dst_hbm.at[pl.ds(wid * EPT, EPT)], dbuf)

    def outer(r, _):
        idx = dbuf[pl.ds(r * 16, 16)]
        plsc.addupdate_scatter(hist, [idx], ones16)
        return _

    lax.fori_loop(0, EPT // 16, outer, None)

    pltpu.sync_copy(hist, histall.at[s])
    plsc.subcore_barrier()
    pltpu.sync_copy(histall.at[:, pl.ds(s * RPT, RPT)], hbuf)

    def red(i, _):
        acc = hbuf[0, pl.ds(i * 16, 16)]
        for k in range(1, NS):
            acc = acc + hbuf[k, pl.ds(i * 16, 16)]
        obuf[pl.ds(i * 16, 16)] = acc
        return _

    lax.fori_loop(0, RPT // 16, red, None)
    pltpu.sync_copy(obuf, deg_out.at[c, pl.ds(s * RPT, RPT)])


_deg_call = pl.kernel(
    _deg_body,
    out_type=jax.ShapeDtypeStruct((NC, N_PAD), jnp.float32),
    mesh=_mesh,
    compiler_params=pltpu.CompilerParams(needs_layout_passes=False),
    scratch_types=[
        pltpu.VMEM((N_PAD,), jnp.float32),
        pltpu.VMEM((EPT,), jnp.int32),
        pltpu.VMEM((NS, RPT), jnp.float32),
        pltpu.VMEM((RPT,), jnp.float32),
        pltpu.VMEM_SHARED((NS, N_PAD), jnp.float32),
    ],
)


def _agg_body(hs_hbm, src_hbm, dst_hbm, part_out,
              sidx, didx, rows, agg, gsem, ssem):
    c = lax.axis_index("c")
    s = lax.axis_index("s")
    wid = c * NS + s

    zeros16 = jnp.zeros((16,), jnp.float32)

    def zero_row(i, _):
        for t in range(NBUF):
            for k in range(D // 16):
                rows[t, i, pl.ds(k * 16, 16)] = zeros16
        return _

    lax.fori_loop(0, CHUNK, zero_row, None)
    for k in range(RPT // CHUNK):
        pltpu.sync_copy(rows.at[0], agg.at[pl.ds(s * RPT + k * CHUNK, CHUNK)])
    plsc.subcore_barrier()

    def gstart(k, t):
        pltpu.async_copy(hs_hbm.at[sidx.at[k]], rows.at[t], gsem)

    def gwait(k, t):
        pltpu.make_async_copy(hs_hbm.at[sidx.at[k]], rows.at[t], gsem).wait()

    def sstart(k, t):
        return pltpu.async_copy(rows.at[t], agg.at[didx.at[k]], ssem,
                                add=True)

    for h in range(NCH // STG):
        row0 = wid * NCH + h * STG
        pltpu.sync_copy(src_hbm.at[pl.ds(row0, STG)], sidx)
        pltpu.sync_copy(dst_hbm.at[pl.ds(row0, STG)], didx)

        for t in range(NBUF):
            gstart(t, t)

        def group(i, _):
            k0 = i * NBUF
            sdescs = []
            for t in range(NBUF):
                gwait(k0 + t, t)
                sdescs.append(sstart(k0 + t, t))
            for t in range(NBUF):
                sdescs[t].wait()

                @pl.when(k0 + NBUF + t < STG)
                def _():
                    gstart(k0 + NBUF + t, t)

            return _

        lax.fori_loop(0, STG // NBUF, group, None)

    plsc.subcore_barrier()

    for k in range(RPT // CHUNK):
        r0 = s * RPT + k * CHUNK
        pltpu.sync_copy(agg.at[pl.ds(r0, CHUNK)], part_out.at[c, pl.ds(r0, CHUNK)])


_agg_call = pl.kernel(
    _agg_body,
    out_type=jax.ShapeDtypeStruct((NC, N_PAD, D), jnp.float32),
    mesh=_mesh,
    scratch_types=[
        pltpu.VMEM((STG, CHUNK), jnp.int32),
        pltpu.VMEM((STG, CHUNK), jnp.int32),
        pltpu.VMEM((NBUF, CHUNK, D), jnp.float32),
        pltpu.VMEM_SHARED((N_PAD, D), jnp.float32),
        pltpu.SemaphoreType.DMA,
        pltpu.SemaphoreType.DMA,
    ],
)

_TC_R = 1024


def _matmul(x_ref, w_ref, h_ref):
    h_ref[...] = jnp.dot(x_ref[...], w_ref[...],
                         preferred_element_type=jnp.float32)


def _phase_a(h_ref, degp_ref, hs_ref, dinv_ref):
    deg = degp_ref[0] + degp_ref[1] + 1.0
    dinv = lax.rsqrt(deg)
    hs_ref[...] = h_ref[...] * dinv[:, None]
    dinv_ref[...] = dinv[:, None]


def _phase_b(agg_ref, hs_ref, dinv_ref, b_ref, w_ref, hs2_ref):
    a = agg_ref[0] + agg_ref[1]
    dinv = dinv_ref[...]
    h1 = jnp.maximum((a + hs_ref[...]) * dinv + b_ref[...], 0.0)
    h2 = jnp.dot(h1, w_ref[...], preferred_element_type=jnp.float32)
    hs2_ref[...] = h2 * dinv


def _phase_c(agg_ref, hs_ref, dinv_ref, b_ref, out_ref):
    a = agg_ref[0] + agg_ref[1]
    out_ref[...] = (a + hs_ref[...]) * dinv_ref[...] + b_ref[...]


_GRID = N_PAD // _TC_R

_matmul_call = pl.pallas_call(
    _matmul,
    grid=(_GRID,),
    in_specs=[
        pl.BlockSpec((_TC_R, D), lambda i: (i, 0)),
        pl.BlockSpec((D, D), lambda i: (0, 0)),
    ],
    out_specs=pl.BlockSpec((_TC_R, D), lambda i: (i, 0)),
    out_shape=jax.ShapeDtypeStruct((N_PAD, D), jnp.float32),
)

_phase_a_call = pl.pallas_call(
    _phase_a,
    grid=(_GRID,),
    in_specs=[
        pl.BlockSpec((_TC_R, D), lambda i: (i, 0)),
        pl.BlockSpec((NC, _TC_R), lambda i: (0, i)),
    ],
    out_specs=[
        pl.BlockSpec((_TC_R, D), lambda i: (i, 0)),
        pl.BlockSpec((_TC_R, 1), lambda i: (i, 0)),
    ],
    out_shape=[
        jax.ShapeDtypeStruct((N_PAD, D), jnp.float32),
        jax.ShapeDtypeStruct((N_PAD, 1), jnp.float32),
    ],
)

_phase_b_call = pl.pallas_call(
    _phase_b,
    grid=(_GRID,),
    in_specs=[
        pl.BlockSpec((NC, _TC_R, D), lambda i: (0, i, 0)),
        pl.BlockSpec((_TC_R, D), lambda i: (i, 0)),
        pl.BlockSpec((_TC_R, 1), lambda i: (i, 0)),
        pl.BlockSpec((D,), lambda i: (0,)),
        pl.BlockSpec((D, D), lambda i: (0, 0)),
    ],
    out_specs=pl.BlockSpec((_TC_R, D), lambda i: (i, 0)),
    out_shape=jax.ShapeDtypeStruct((N_PAD, D), jnp.float32),
)

_TC_RC = 2000

_phase_c_call = pl.pallas_call(
    _phase_c,
    grid=(N // _TC_RC,),
    in_specs=[
        pl.BlockSpec((NC, _TC_RC, D), lambda i: (0, i, 0)),
        pl.BlockSpec((_TC_RC, D), lambda i: (i, 0)),
        pl.BlockSpec((_TC_RC, 1), lambda i: (i, 0)),
        pl.BlockSpec((D,), lambda i: (0,)),
    ],
    out_specs=pl.BlockSpec((_TC_RC, D), lambda i: (i, 0)),
    out_shape=jax.ShapeDtypeStruct((N, D), jnp.float32),
)


@jax.jit
def kernel(x, edge_index, W1, b1, W2, b2):
    src = edge_index[0]
    dst = edge_index[1]
    src_p = jnp.concatenate([src, _PAD_SRC]).reshape(TOT_CH, CHUNK)
    dst_p = jnp.concatenate([dst, _PAD_DST]).reshape(TOT_CH, CHUNK)
    x_p = jnp.pad(x, ((0, N_PAD - N), (0, 0)))

    h1 = _matmul_call(x_p, W1)
    degp = _deg_call(dst)
    hs1, dinv = _phase_a_call(h1, degp)
    agg1 = _agg_call(hs1, src_p, dst_p)
    hs2 = _phase_b_call(agg1, hs1, dinv, b1, W2)
    agg2 = _agg_call(hs2, src_p, dst_p)
    return _phase_c_call(agg2, hs2, dinv, b2)
